# baseline (speedup 1.0000x reference)
import functools

import jax
import jax.numpy as jnp
from jax import lax
from jax.experimental import pallas as pl
from jax.experimental.pallas import tpu as pltpu
from jax.experimental.pallas import tpu_sc as plsc

B = 16
S = 2048
D_IN = 512
D_OUT = 512


G = 2


def _body(x_ref, wsT_ref, wp_ref, bp_ref, o_ref, s_scr):
    xb = x_ref[...]
    x3 = xb.reshape(G * S // 128, 128, D_IN)
    s_scr[...] = jnp.sum(x3 * wsT_ref[...][None], axis=2)
    s = s_scr[...].reshape(G, S // 128, 128)
    mean = jnp.sum(s, axis=(1, 2), keepdims=True) / S
    sumsq = jnp.sum(s * s, axis=(1, 2), keepdims=True)
    var = (sumsq - S * mean * mean) / (S - 1)
    a = 1.0 / (jnp.sqrt(var) + 1e-7)
    e = jnp.exp(s * a - mean * a)
    w = e.reshape(1, G * S)
    col_g = jax.lax.broadcasted_iota(jnp.int32, (G, G * S), 1) // S
    row_g = jax.lax.broadcasted_iota(jnp.int32, (G, G * S), 0)
    w_bd = jnp.where(col_g == row_g, w, 0.0)
    denom = jnp.sum(e, axis=(1, 2)).reshape(G, 1)
    pooled = jnp.dot(w_bd, xb, preferred_element_type=jnp.float32) / denom
    o_ref[...] = (
        jnp.dot(pooled, wp_ref[...], preferred_element_type=jnp.float32)
        + bp_ref[...]
    )[None]


@jax.jit
def _run(x, wsT, W_proj, bp):
    return pl.pallas_call(
        _body,
        grid=(B // G,),
        in_specs=[
            pl.BlockSpec((G * S, D_IN), lambda b: (b, 0)),
            pl.BlockSpec((1, D_IN), lambda b: (0, 0)),
            pl.BlockSpec((D_IN, D_OUT), lambda b: (0, 0)),
            pl.BlockSpec((1, D_OUT), lambda b: (0, 0)),
        ],
        out_specs=pl.BlockSpec((1, G, D_OUT), lambda b: (b, 0, 0)),
        out_shape=jax.ShapeDtypeStruct((B // G, G, D_OUT), jnp.float32),
        scratch_shapes=[pltpu.VMEM((G * S // 128, 128), jnp.float32)],
        compiler_params=pltpu.CompilerParams(
            dimension_semantics=("arbitrary",),
        ),
    )(x, wsT, W_proj, bp)


_NW = 32
_RPW = (B * S) // _NW
_CHUNK = 128


def _lane_perm(v, idx):
    dn = lax.GatherDimensionNumbers(
        offset_dims=(), collapsed_slice_dims=(0,), start_index_map=(0,))
    return lax.gather(
        v, idx[:, None], dn, (1,),
        mode=lax.GatherScatterMode.PROMISE_IN_BOUNDS)


def _sc_scores_body(x_hbm, ws_hbm, out_hbm, xt, st, wsv):
    wid = lax.axis_index("s") * 2 + lax.axis_index("c")
    base = wid * _RPW
    pltpu.sync_copy(ws_hbm, wsv)

    for chunk in range(_RPW // _CHUNK):
        pltpu.sync_copy(
            x_hbm.at[pl.ds((base + chunk * _CHUNK) * D_IN, _CHUNK * D_IN)],
            xt)

        lane = lax.iota(jnp.int32, 16)

        def group_body(g, _):
            gbase = pl.multiple_of(g * 16 * D_IN, 16 * D_IN)
            svec = jnp.zeros((16,), jnp.float32)
            for r in range(16):
                acc = jnp.zeros((16,), jnp.float32)
                for k in range(D_IN // 16):
                    acc = acc + (
                        xt[pl.ds(gbase + r * D_IN + 16 * k, 16)]
                        * wsv[pl.ds(16 * k, 16)]
                    )
                for sh in (8, 4, 2, 1):
                    acc = acc + _lane_perm(acc, lane ^ sh)
                svec = jnp.where(lane == r, acc, svec)
            soff = pl.multiple_of(chunk * _CHUNK + g * 16, 16)
            st[pl.ds(soff, 16)] = svec
            return 0

        lax.fori_loop(0, _CHUNK // 16, group_body, 0)

    pltpu.sync_copy(st, out_hbm.at[pl.ds(base, _RPW)])


@jax.jit
def _sc_scores(x, ws):
    mesh = plsc.VectorSubcoreMesh(core_axis_name="c", subcore_axis_name="s")
    f = functools.partial(
        pl.kernel,
        mesh=mesh,
        out_type=jax.ShapeDtypeStruct((B * S,), jnp.float32),
        scratch_types=[
            pltpu.VMEM((_CHUNK * D_IN,), jnp.float32),
            pltpu.VMEM((_RPW,), jnp.float32),
            pltpu.VMEM((D_IN,), jnp.float32),
        ],
    )(_sc_scores_body)
    return f(x, ws)


def _body_h(x_ref, sc_ref, wp_ref, bp_ref, o_ref):
    xb = x_ref[...]
    s = sc_ref[...].reshape(G, S // 128, 128)
    mean = jnp.sum(s, axis=(1, 2), keepdims=True) / S
    sumsq = jnp.sum(s * s, axis=(1, 2), keepdims=True)
    var = (sumsq - S * mean * mean) / (S - 1)
    a = 1.0 / (jnp.sqrt(var) + 1e-7)
    e = jnp.exp(s * a - mean * a)
    w = e.reshape(1, G * S)
    col_g = jax.lax.broadcasted_iota(jnp.int32, (G, G * S), 1) // S
    row_g = jax.lax.broadcasted_iota(jnp.int32, (G, G * S), 0)
    w_bd = jnp.where(col_g == row_g, w, 0.0)
    denom = jnp.sum(e, axis=(1, 2)).reshape(G, 1)
    pooled = jnp.dot(w_bd, xb, preferred_element_type=jnp.float32) / denom
    o_ref[...] = (
        jnp.dot(pooled, wp_ref[...], preferred_element_type=jnp.float32)
        + bp_ref[...]
    )[None]


@jax.jit
def _run_h(x, scores, W_proj, bp):
    return pl.pallas_call(
        _body_h,
        grid=(B // G,),
        in_specs=[
            pl.BlockSpec((G * S, D_IN), lambda b: (b, 0)),
            pl.BlockSpec((1, 1, G * S), lambda b: (b, 0, 0)),
            pl.BlockSpec((D_IN, D_OUT), lambda b: (0, 0)),
            pl.BlockSpec((1, D_OUT), lambda b: (0, 0)),
        ],
        out_specs=pl.BlockSpec((1, G, D_OUT), lambda b: (b, 0, 0)),
        out_shape=jax.ShapeDtypeStruct((B // G, G, D_OUT), jnp.float32),
        compiler_params=pltpu.CompilerParams(
            dimension_semantics=("arbitrary",),
        ),
    )(x, scores, W_proj, bp)


def kernel(x, W_proj, b_proj, W_score, b_score, graph_size_list, edge_list):
    bp = b_proj.reshape(1, D_OUT)
    scores = _sc_scores(x.reshape(B * S * D_IN), W_score.reshape(D_IN))
    return _run_h(
        x, scores.reshape(B // G, 1, G * S), W_proj, bp
    ).reshape(B, D_OUT)


# speedup vs baseline: 7.0878x; 7.0878x over previous
import jax
import jax.numpy as jnp
from jax.experimental import pallas as pl
from jax.experimental.pallas import tpu as pltpu

B = 16
S = 2048
D_IN = 512
D_OUT = 512


def _body(xa_ref, xb_ref, wsT_ref, wp_ref, bp_ref, o_ref, sa_scr, sb_scr):
    def one(x_ref, s_scr):
        xg = x_ref[...]
        x3 = xg.reshape(S // 128, 128, D_IN)
        s_scr[...] = jnp.sum(x3 * wsT_ref[...][None], axis=2)
        s = s_scr[...]
        mean = jnp.sum(s) / S
        sumsq = jnp.sum(s * s)
        var = (sumsq - S * mean * mean) / (S - 1)
        a = 1.0 / (jnp.sqrt(var) + 1e-7)
        e = jnp.exp(s * a - mean * a)
        w = e.reshape(1, S)
        pooled = jnp.dot(w, xg, preferred_element_type=jnp.float32)
        return pooled / jnp.sum(e)

    pooled = jnp.concatenate([one(xa_ref, sa_scr), one(xb_ref, sb_scr)], 0)
    o_ref[...] = (
        jnp.dot(pooled, wp_ref[...], preferred_element_type=jnp.float32)
        + bp_ref[...]
    )[None]


@jax.jit
def _run(x, wsT, W_proj, bp):
    return pl.pallas_call(
        _body,
        grid=(B // 2,),
        in_specs=[
            pl.BlockSpec((S, D_IN), lambda b: (2 * b, 0)),
            pl.BlockSpec((S, D_IN), lambda b: (2 * b + 1, 0)),
            pl.BlockSpec((1, D_IN), lambda b: (0, 0)),
            pl.BlockSpec((D_IN, D_OUT), lambda b: (0, 0)),
            pl.BlockSpec((1, D_OUT), lambda b: (0, 0)),
        ],
        out_specs=pl.BlockSpec((1, 2, D_OUT), lambda b: (b, 0, 0)),
        out_shape=jax.ShapeDtypeStruct((B // 2, 2, D_OUT), jnp.float32),
        scratch_shapes=[
            pltpu.VMEM((S // 128, 128), jnp.float32),
            pltpu.VMEM((S // 128, 128), jnp.float32),
        ],
        compiler_params=pltpu.CompilerParams(
            dimension_semantics=("arbitrary",),
        ),
    )(x, x, wsT, W_proj, bp)


def kernel(x, W_proj, b_proj, W_score, b_score, graph_size_list, edge_list):
    wsT = W_score.reshape(1, D_IN)
    bp = b_proj.reshape(1, D_OUT)
    return _run(x, wsT, W_proj, bp).reshape(B, D_OUT)
